# baseline (speedup 1.0000x reference)
import jax
import jax.numpy as jnp
from jax.experimental import pallas as pl
from jax.experimental.pallas import tpu as pltpu

C = 16
R = 2
GR = 8

_PREC = jax.lax.Precision.DEFAULT

_NGU = C * (GR + R)
_NSEL = _NGU + C + C * R


def _fused(x_ref, w_ref, b_ref, wall_ref, o_ref, x1_ref, y_ref, d2u_ref):
    m = o_ref.shape[0]
    nblk = pl.num_programs(0) - 1
    i = pl.program_id(0)
    off = (i % 2) * m
    offp = m - off

    @pl.when(i == 0)
    def _compute_d2u():
        ud = wall_ref[C * GR:_NGU, :] * wall_ref[_NSEL:, :]
        ones_row = jnp.full((1, wall_ref.shape[1]), 1.0, dtype=jnp.float32)
        d2u = jax.lax.dot_general(
            ones_row, ud, (((1,), (1,)), ((), ())),
            precision=jax.lax.Precision.HIGHEST,
            preferred_element_type=jnp.float32)
        d2u_ref[...] = jnp.concatenate(
            [jnp.zeros((1, C), jnp.float32), d2u], axis=1)

    @pl.when(i < nblk)
    def _produce():
        x1 = jax.lax.dot_general(
            x_ref[...], w_ref[...], (((1,), (1,)), ((), ())),
            precision=_PREC, preferred_element_type=jnp.float32)
        x1 = x1 + b_ref[...]
        x1_ref[pl.ds(off, m), :] = x1
        y_ref[pl.ds(off, m), :] = jax.lax.dot_general(
            x1, wall_ref[:_NGU, :], (((1,), (1,)), ((), ())),
            precision=_PREC, preferred_element_type=jnp.float32)

    @pl.when(i > 0)
    def _consume():
        x1 = x1_ref[pl.ds(offp, m), :]
        y = y_ref[pl.ds(offp, m), :]
        proj = y[:, :C * GR]
        P = y[:, C * GR:]
        proj2 = proj * proj
        srow = jax.lax.broadcasted_iota(jnp.int32, (C * GR, C), 0) // GR
        scol = jax.lax.broadcasted_iota(jnp.int32, (C * GR, C), 1)
        sel = (srow == scol).astype(jnp.float32)
        score = jax.lax.dot_general(
            proj2, sel, (((1,), (0,)), ((), ())),
            precision=jax.lax.Precision.HIGHEST,
            preferred_element_type=jnp.float32)

        idx = jnp.argmax(score, axis=-1)
        smax = jnp.max(score, axis=-1, keepdims=True)
        sg = jax.nn.sigmoid(smax)

        lane48 = jax.lax.broadcasted_iota(jnp.int32, (m, C * (1 + R)), 1)
        c_of = jnp.where(lane48 < C, lane48, (lane48 - C) // R)
        g48 = (c_of == idx[:, None]).astype(jnp.float32)
        Ppad = jnp.concatenate(
            [jnp.full((m, C), 1.0, jnp.float32), P], axis=1)
        selcat = g48 * (Ppad - d2u_ref[...])
        biasmod = jax.lax.dot_general(
            selcat, wall_ref[_NGU:_NSEL, :], (((1,), (0,)), ((), ())),
            precision=_PREC, preferred_element_type=jnp.float32)

        o_ref[...] = x1 + sg * (biasmod - x1)


def kernel(x, W_org, b_org, W_gate, lora_update, lora_degen, bias_w,
           debias_w):
    B, T, D = x.shape
    BT = B * T
    M = 256
    assert BT % M == 0
    nblk = BT // M

    xf = x.reshape(BT, D)
    b2 = b_org.reshape(1, D)
    Wall = jnp.concatenate([
        W_gate.transpose(0, 2, 1).reshape(C * GR, D),
        lora_update.transpose(0, 2, 1).reshape(C * R, D),
        bias_w,
        lora_degen.transpose(0, 2, 1).reshape(C * R, D),
        jnp.repeat(debias_w, R, axis=0),
    ], axis=0)

    out = pl.pallas_call(
        _fused,
        grid=(nblk + 1,),
        in_specs=[
            pl.BlockSpec((M, D), lambda i: (jnp.minimum(i, nblk - 1), 0)),
            pl.BlockSpec((D, D), lambda i: (0, 0)),
            pl.BlockSpec((1, D), lambda i: (0, 0)),
            pl.BlockSpec((_NSEL + C * R, D), lambda i: (0, 0)),
        ],
        out_specs=pl.BlockSpec(
            (M, D), lambda i: (jnp.maximum(i - 1, 0), 0)),
        out_shape=jax.ShapeDtypeStruct((BT, D), jnp.float32),
        scratch_shapes=[
            pltpu.VMEM((2 * M, D), jnp.float32),
            pltpu.VMEM((2 * M, _NGU), jnp.float32),
            pltpu.VMEM((1, C * (1 + R)), jnp.float32),
        ],
        compiler_params=pltpu.CompilerParams(
            dimension_semantics=("arbitrary",)),
    )(xf, W_org, b2, Wall)
    return out.reshape(B, T, D)


# speedup vs baseline: 1.0706x; 1.0706x over previous
import jax
import jax.numpy as jnp
from jax.experimental import pallas as pl
from jax.experimental.pallas import tpu as pltpu

C = 16
R = 2
GR = 8

_PREC = jax.lax.Precision.DEFAULT

_NGU = C * (GR + R)
_NSEL = _NGU + C + C * R


def _fused(x_ref, w_ref, b_ref, wall_ref, o_ref, x1_ref, y_ref, d2u_ref):
    m = o_ref.shape[0]
    nblk = pl.num_programs(0) - 1
    i = pl.program_id(0)
    off = (i % 2) * m
    offp = m - off

    @pl.when(i == 0)
    def _compute_d2u():
        ud = wall_ref[C * GR:_NGU, :] * wall_ref[_NSEL:, :]
        ones_row = jnp.full((1, wall_ref.shape[1]), 1.0, dtype=jnp.float32)
        d2u = jax.lax.dot_general(
            ones_row, ud, (((1,), (1,)), ((), ())),
            precision=jax.lax.Precision.HIGHEST,
            preferred_element_type=jnp.float32)
        d2u_ref[...] = jnp.concatenate(
            [jnp.zeros((1, C), jnp.float32), d2u], axis=1)

    @pl.when(i < nblk)
    def _produce():
        x1 = jax.lax.dot_general(
            x_ref[...], w_ref[...], (((1,), (1,)), ((), ())),
            precision=_PREC, preferred_element_type=jnp.float32)
        x1 = x1 + b_ref[...]
        x1_ref[pl.ds(off, m), :] = x1
        y_ref[pl.ds(off, m), :] = jax.lax.dot_general(
            x1, wall_ref[:_NGU, :], (((1,), (1,)), ((), ())),
            precision=_PREC, preferred_element_type=jnp.float32)

    @pl.when(i > 0)
    def _consume():
        x1 = x1_ref[pl.ds(offp, m), :]
        y = y_ref[pl.ds(offp, m), :]
        proj = y[:, :C * GR]
        P = y[:, C * GR:]
        proj2 = proj * proj
        srow = jax.lax.broadcasted_iota(jnp.int32, (C * GR, C), 0) // GR
        scol = jax.lax.broadcasted_iota(jnp.int32, (C * GR, C), 1)
        sel = (srow == scol).astype(jnp.float32)
        score = jax.lax.dot_general(
            proj2, sel, (((1,), (0,)), ((), ())),
            precision=jax.lax.Precision.HIGHEST,
            preferred_element_type=jnp.float32)

        idx = jnp.argmax(score, axis=-1)
        smax = jnp.max(score, axis=-1, keepdims=True)
        sg = jax.nn.sigmoid(smax)

        lane48 = jax.lax.broadcasted_iota(jnp.int32, (m, C * (1 + R)), 1)
        c_of = jnp.where(lane48 < C, lane48, (lane48 - C) // R)
        g48 = (c_of == idx[:, None]).astype(jnp.float32)
        Ppad = jnp.concatenate(
            [jnp.full((m, C), 1.0, jnp.float32), P], axis=1)
        selcat = g48 * (Ppad - d2u_ref[...])
        biasmod = jax.lax.dot_general(
            selcat, wall_ref[_NGU:_NSEL, :], (((1,), (0,)), ((), ())),
            precision=_PREC, preferred_element_type=jnp.float32)

        o_ref[...] = x1 + sg * (biasmod - x1)


def kernel(x, W_org, b_org, W_gate, lora_update, lora_degen, bias_w,
           debias_w):
    B, T, D = x.shape
    BT = B * T
    M = 512
    assert BT % M == 0
    nblk = BT // M

    xf = x.reshape(BT, D)
    b2 = b_org.reshape(1, D)
    Wall = jnp.concatenate([
        W_gate.transpose(0, 2, 1).reshape(C * GR, D),
        lora_update.transpose(0, 2, 1).reshape(C * R, D),
        bias_w,
        lora_degen.transpose(0, 2, 1).reshape(C * R, D),
        jnp.repeat(debias_w, R, axis=0),
    ], axis=0)

    out = pl.pallas_call(
        _fused,
        grid=(nblk + 1,),
        in_specs=[
            pl.BlockSpec((M, D), lambda i: (jnp.minimum(i, nblk - 1), 0)),
            pl.BlockSpec((D, D), lambda i: (0, 0)),
            pl.BlockSpec((1, D), lambda i: (0, 0)),
            pl.BlockSpec((_NSEL + C * R, D), lambda i: (0, 0)),
        ],
        out_specs=pl.BlockSpec(
            (M, D), lambda i: (jnp.maximum(i - 1, 0), 0)),
        out_shape=jax.ShapeDtypeStruct((BT, D), jnp.float32),
        scratch_shapes=[
            pltpu.VMEM((2 * M, D), jnp.float32),
            pltpu.VMEM((2 * M, _NGU), jnp.float32),
            pltpu.VMEM((1, C * (1 + R)), jnp.float32),
        ],
        compiler_params=pltpu.CompilerParams(
            dimension_semantics=("arbitrary",)),
    )(xf, W_org, b2, Wall)
    return out.reshape(B, T, D)


# speedup vs baseline: 1.0743x; 1.0035x over previous
import jax
import jax.numpy as jnp
from jax.experimental import pallas as pl
from jax.experimental.pallas import tpu as pltpu

C = 16
R = 2
GR = 8

_PREC = jax.lax.Precision.DEFAULT

_NGU = C * (GR + R)
_NSEL = _NGU + C + C * R


def _fused(x_ref, w_ref, b_ref, wall_ref, o_ref, x1_ref, y_ref, d2u_ref):
    m = o_ref.shape[0]
    nblk = pl.num_programs(0) - 1
    i = pl.program_id(0)
    off = (i % 2) * m
    offp = m - off

    @pl.when(i == 0)
    def _compute_d2u():
        ud = wall_ref[C * GR:_NGU, :] * wall_ref[_NSEL:, :]
        ones_row = jnp.full((1, wall_ref.shape[1]), 1.0, dtype=jnp.float32)
        d2u = jax.lax.dot_general(
            ones_row, ud, (((1,), (1,)), ((), ())),
            precision=jax.lax.Precision.HIGHEST,
            preferred_element_type=jnp.float32)
        d2u_ref[...] = jnp.concatenate(
            [jnp.zeros((1, C), jnp.float32), d2u], axis=1)

    @pl.when(i < nblk)
    def _produce():
        x1 = jax.lax.dot_general(
            x_ref[...], w_ref[...], (((1,), (1,)), ((), ())),
            precision=_PREC, preferred_element_type=jnp.float32)
        x1 = x1 + b_ref[...]
        x1_ref[pl.ds(off, m), :] = x1
        y_ref[pl.ds(off, m), :] = jax.lax.dot_general(
            x1, wall_ref[:_NGU, :], (((1,), (1,)), ((), ())),
            precision=_PREC, preferred_element_type=jnp.float32)

    @pl.when(i > 0)
    def _consume():
        x1 = x1_ref[pl.ds(offp, m), :]
        y = y_ref[pl.ds(offp, m), :]
        proj = y[:, :C * GR]
        P = y[:, C * GR:]
        proj2 = proj * proj
        srow = jax.lax.broadcasted_iota(jnp.int32, (C * GR, C), 0) // GR
        scol = jax.lax.broadcasted_iota(jnp.int32, (C * GR, C), 1)
        sel = (srow == scol).astype(jnp.float32)
        score = jax.lax.dot_general(
            proj2, sel, (((1,), (0,)), ((), ())),
            precision=jax.lax.Precision.HIGHEST,
            preferred_element_type=jnp.float32)

        idx = jnp.argmax(score, axis=-1)
        smax = jnp.max(score, axis=-1, keepdims=True)
        sg = jax.nn.sigmoid(smax)

        lane48 = jax.lax.broadcasted_iota(jnp.int32, (m, C * (1 + R)), 1)
        c_of = jnp.where(lane48 < C, lane48, (lane48 - C) % C)
        g48 = (c_of == idx[:, None]).astype(jnp.float32)
        Ppad = jnp.concatenate(
            [jnp.full((m, C), 1.0, jnp.float32), P], axis=1)
        selcat = g48 * (Ppad - d2u_ref[...])
        biasmod = jax.lax.dot_general(
            selcat, wall_ref[_NGU:_NSEL, :], (((1,), (0,)), ((), ())),
            precision=_PREC, preferred_element_type=jnp.float32)

        o_ref[...] = x1 + sg * (biasmod - x1)


def kernel(x, W_org, b_org, W_gate, lora_update, lora_degen, bias_w,
           debias_w):
    B, T, D = x.shape
    BT = B * T
    M = 512
    assert BT % M == 0
    nblk = BT // M

    xf = x.reshape(BT, D)
    b2 = b_org.reshape(1, D)
    Wall = jnp.concatenate([
        W_gate.transpose(0, 2, 1).reshape(C * GR, D),
        lora_update.transpose(2, 0, 1).reshape(C * R, D),
        bias_w,
        lora_degen.transpose(2, 0, 1).reshape(C * R, D),
        debias_w,
        debias_w,
    ], axis=0)

    out = pl.pallas_call(
        _fused,
        grid=(nblk + 1,),
        in_specs=[
            pl.BlockSpec((M, D), lambda i: (jnp.minimum(i, nblk - 1), 0)),
            pl.BlockSpec((D, D), lambda i: (0, 0)),
            pl.BlockSpec((1, D), lambda i: (0, 0)),
            pl.BlockSpec((_NSEL + C * R, D), lambda i: (0, 0)),
        ],
        out_specs=pl.BlockSpec(
            (M, D), lambda i: (jnp.maximum(i - 1, 0), 0)),
        out_shape=jax.ShapeDtypeStruct((BT, D), jnp.float32),
        scratch_shapes=[
            pltpu.VMEM((2 * M, D), jnp.float32),
            pltpu.VMEM((2 * M, _NGU), jnp.float32),
            pltpu.VMEM((1, C * (1 + R)), jnp.float32),
        ],
        compiler_params=pltpu.CompilerParams(
            dimension_semantics=("arbitrary",)),
    )(xf, W_org, b2, Wall)
    return out.reshape(B, T, D)
